# baseline (speedup 1.0000x reference)
import functools
import jax
import jax.numpy as jnp
from jax import lax
from jax.experimental import pallas as pl
from jax.experimental.pallas import tpu as pltpu
from jax.experimental.pallas import tpu_sc as plsc

THR = 0.05


def _sc_scatter(gidx, val, n, r_chunk=256):
    ne, epad = gidx.shape
    nsub = 16
    ncores = 2
    pt = epad // nsub
    nb = pt // 128
    half = n // ncores
    nch = half // r_chunk
    chunk_words = r_chunk * n
    slice_words = chunk_words // nsub
    zwords = 8192
    mesh = plsc.VectorSubcoreMesh(core_axis_name="c", subcore_axis_name="s")

    @functools.partial(
        pl.kernel, mesh=mesh,
        out_type=jax.ShapeDtypeStruct((ne, n * n), jnp.float32),
        scratch_types=[
            pltpu.VMEM((pt,), jnp.int32),
            pltpu.VMEM((pt,), jnp.float32),
            pltpu.VMEM((nb, 128), jnp.int32),
            pltpu.VMEM((nb, 128), jnp.float32),
            pltpu.VMEM((zwords,), jnp.float32),
            pltpu.VMEM_SHARED((chunk_words,), jnp.float32),
            pltpu.SemaphoreType.DMA,
        ],
    )
    def scat(gidx_hbm, val_hbm, out_hbm, gidx_v, val_v,
             idx2d, sval2d, zeros_v, shared, sem):
        cid = lax.axis_index("c")
        sid = lax.axis_index("s")

        def zinit(i, carry):
            zeros_v[pl.ds(i * 16, 16)] = jnp.zeros((16,), jnp.float32)
            return carry
        lax.fori_loop(0, zwords // 16, zinit, 0)

        def plane_body(e, carry):
            pltpu.sync_copy(gidx_hbm.at[e, pl.ds(sid * pt, pt)], gidx_v)
            pltpu.sync_copy(val_hbm.at[e, pl.ds(sid * pt, pt)], val_v)

            def chunk_body(ch, carry2):
                base = (cid * half + ch * r_chunk) * n
                for z in range(slice_words // zwords):
                    pltpu.sync_copy(
                        zeros_v,
                        shared.at[pl.ds(sid * slice_words + z * zwords,
                                        zwords)])
                plsc.subcore_barrier()

                def scan_body(b, carry3):
                    for g in range(8):
                        p = b * 128 + g * 16
                        lr = gidx_v[pl.ds(p, 16)] - base
                        v = val_v[pl.ds(p, 16)]
                        inr = (lr >= 0) & (lr < chunk_words)
                        idx2d[b, pl.ds(g * 16, 16)] = lr & (chunk_words - 1)
                        sval2d[b, pl.ds(g * 16, 16)] = jnp.where(inr, v, 0.0)
                    return carry3
                lax.fori_loop(0, nb, scan_body, 0)

                copies = [pltpu.async_copy(sval2d.at[b],
                                           shared.at[idx2d.at[b]],
                                           sem, add=True)
                          for b in range(nb)]
                for cpy in copies:
                    cpy.wait()
                plsc.subcore_barrier()

                pltpu.sync_copy(shared.at[pl.ds(sid * slice_words,
                                                slice_words)],
                                out_hbm.at[e, pl.ds(base + sid * slice_words,
                                                    slice_words)])
                return carry2
            lax.fori_loop(0, nch, chunk_body, 0)
            return carry
        lax.fori_loop(0, ne, plane_body, 0)

    return scat(gidx, val)


def _combine_body(w_ref, a_ref, ra_ref, rb_ref, rb2_ref, *, ne, nc):
    a = a_ref[...].reshape(ne, -1)
    r = jnp.dot(w_ref[...], a, preferred_element_type=jnp.float32)
    for x, oref in enumerate((ra_ref, rb_ref, rb2_ref)):
        blk = r[x * nc:(x + 1) * nc].reshape(oref.shape)
        oref[...] = blk.astype(oref.dtype)


def _combine(w, A, bi=256, bj=2048):
    ne, n, _ = A.shape
    nc = w.shape[1]
    bi = min(bi, n)
    bj = min(bj, n)
    grid = (n // bi, n // bj)
    out_sd = jax.ShapeDtypeStruct((nc, n, n), jnp.bfloat16)
    out_spec = pl.BlockSpec((nc, bi, bj), lambda i, j: (0, i, j))
    return pl.pallas_call(
        functools.partial(_combine_body, ne=ne, nc=nc),
        grid=grid,
        in_specs=[
            pl.BlockSpec((3 * nc, ne), lambda i, j: (0, 0)),
            pl.BlockSpec((ne, bi, bj), lambda i, j: (0, i, j)),
        ],
        out_specs=[out_spec, out_spec, out_spec],
        out_shape=[out_sd, out_sd, out_sd],
    )(w.reshape(3 * nc, ne), A)


def _mm_body(lhs_ref, rhs_ref, out_ref, rs_ref, *, bi, bk, nk, thr):
    c, i, j = pl.program_id(0), pl.program_id(1), pl.program_id(2)
    acc = jnp.dot(lhs_ref[0], rhs_ref[0], preferred_element_type=jnp.float32)
    h = jnp.where(acc > thr, acc, 0.0)
    out_ref[0] = h.astype(out_ref.dtype)

    @pl.when((c == 0) & (i == 0) & (j == 0))
    def _():
        rs_ref[...] = jnp.zeros_like(rs_ref)

    ones = jnp.ones((h.shape[1],), jnp.float32)
    rs_ref[pl.ds(c, 1), pl.ds(i * bi, bi)] += jnp.dot(
        h, ones, preferred_element_type=jnp.float32)[None, :]


def _mm_scaled_body(d_ref, lhs_ref, rhs_ref, out_ref, rs_ref, *, bi, bk, nk,
                    thr):
    c, i, j = pl.program_id(0), pl.program_id(1), pl.program_id(2)

    acc = jnp.dot(lhs_ref[0], rhs_ref[0], preferred_element_type=jnp.float32)
    d = d_ref[pl.ds(c, 1), pl.ds(i * bi, bi)][0]
    dinv = jnp.where(d == 0.0, 0.0, 1.0 / jnp.where(d == 0.0, 1.0, d))
    acc = acc * dinv[:, None]
    h = jnp.where(acc > thr, acc, 0.0)
    out_ref[0] = h.astype(out_ref.dtype)

    @pl.when((c == 0) & (i == 0) & (j == 0))
    def _():
        rs_ref[...] = jnp.zeros_like(rs_ref)

    ones = jnp.ones((h.shape[1],), jnp.float32)
    rs_ref[pl.ds(c, 1), pl.ds(i * bi, bi)] += jnp.dot(
        h, ones, preferred_element_type=jnp.float32)[None, :]


def _mm_threshold(lhs, rhs, d=None, bi=1024, bj=1024, bk=512):
    nc, n, _ = lhs.shape
    bi, bj, bk = min(bi, n), min(bj, n), min(bk, n)
    grid = (nc, n // bi, n // bj)
    nk = n // bk
    out_shape = [jax.ShapeDtypeStruct((nc, n, n), jnp.bfloat16),
                 jax.ShapeDtypeStruct((nc, n), jnp.float32)]
    out_specs = [pl.BlockSpec((1, bi, bj), lambda c, i, j: (c, i, j)),
                 pl.BlockSpec((nc, n), lambda c, i, j: (0, 0))]
    lhs_spec = pl.BlockSpec((1, bi, n), lambda c, i, j: (c, i, 0))
    rhs_spec = pl.BlockSpec((1, n, bj), lambda c, i, j: (c, 0, j))
    if d is None:
        return pl.pallas_call(
            functools.partial(_mm_body, bi=bi, bk=bk, nk=nk, thr=THR),
            grid=grid,
            in_specs=[lhs_spec, rhs_spec],
            out_specs=out_specs,
            out_shape=out_shape,
        )(lhs, rhs)
    d_spec = pl.BlockSpec((nc, n), lambda c, i, j: (0, 0))
    return pl.pallas_call(
        functools.partial(_mm_scaled_body, bi=bi, bk=bk, nk=nk, thr=THR),
        grid=grid,
        in_specs=[d_spec, lhs_spec, rhs_spec],
        out_specs=out_specs,
        out_shape=out_shape,
    )(d, lhs, rhs)


def _gcn_body(d_ref, lhs_ref, xw_ref, b_ref, out_ref, *, bi, bk, nk):
    c, i = pl.program_id(0), pl.program_id(1)

    acc = jnp.dot(lhs_ref[0], xw_ref[...],
                  preferred_element_type=jnp.float32)
    d = d_ref[pl.ds(c, 1), pl.ds(i * bi, bi)][0]
    dinv = jnp.where(d == 0.0, 0.0, 1.0 / jnp.where(d == 0.0, 1.0, d))
    acc = acc * dinv[:, None] + b_ref[...]
    out_ref[0] = jnp.maximum(acc, 0.0)


def _gcn(d, H, XW, b, bi=1024, bk=512):
    nc, n, _ = H.shape
    wout = XW.shape[1]
    bi, bk = min(bi, n), min(bk, n)
    grid = (nc, n // bi)
    return pl.pallas_call(
        functools.partial(_gcn_body, bi=bi, bk=bk, nk=n // bk),
        grid=grid,
        in_specs=[
            pl.BlockSpec((nc, n), lambda c, i: (0, 0)),
            pl.BlockSpec((1, bi, n), lambda c, i: (c, i, 0)),
            pl.BlockSpec((n, wout), lambda c, i: (0, 0)),
            pl.BlockSpec((1, wout), lambda c, i: (0, 0)),
        ],
        out_specs=pl.BlockSpec((1, bi, wout), lambda c, i: (c, i, 0)),
        out_shape=jax.ShapeDtypeStruct((nc, n, wout), jnp.float32),
    )(d, H, XW, b.reshape(1, wout))


def _xw_body(x_ref, w_ref, out_ref):
    out_ref[...] = jnp.dot(x_ref[...], w_ref[...],
                           preferred_element_type=jnp.float32
                           ).astype(out_ref.dtype)


def _xw(X, W):
    return pl.pallas_call(
        _xw_body,
        out_shape=jax.ShapeDtypeStruct((X.shape[0], W.shape[1]),
                                       jnp.bfloat16),
    )(X, W)


def kernel(A_edge_index, A_edge_value, X, l0_w1, l0_w2, l1_w1, gcn_w, gcn_b):
    ne, _, E = A_edge_index.shape
    n = X.shape[0]
    wa = jax.nn.softmax(l0_w1, axis=1)
    wb = jax.nn.softmax(l0_w2, axis=1)
    wb2 = jax.nn.softmax(l1_w1, axis=1)
    w = jnp.stack([wa, wb, wb2])

    nsub = 16
    pt = ((E + nsub * 128 - 1) // (nsub * 128)) * 128
    epad = nsub * pt
    padw = ((0, 0), (0, epad - E))
    gidx = A_edge_index[:, 0, :] * n + A_edge_index[:, 1, :]
    gidx = jnp.pad(gidx, padw, constant_values=n * n)
    val = jnp.pad(A_edge_value, padw, constant_values=0.0)
    A = _sc_scatter(gidx.astype(jnp.int32), val, n).reshape(ne, n, n)

    Ra, Rb, Rb2 = _combine(w, A)
    H1, D1 = _mm_threshold(Ra, Rb)
    H2, D2 = _mm_threshold(H1, Rb2, d=D1)
    XW = _xw(X, gcn_w)
    out = _gcn(D2, H2, XW, gcn_b)
    nc, _, wout = out.shape
    return out.transpose(1, 0, 2).reshape(n, nc * wout)


# speedup vs baseline: 1.0562x; 1.0562x over previous
import functools
import jax
import jax.numpy as jnp
from jax import lax
from jax.experimental import pallas as pl
from jax.experimental.pallas import tpu as pltpu
from jax.experimental.pallas import tpu_sc as plsc

THR = 0.05


def _sc_scatter(gidx, val, n, r_chunk=256):
    ne, epad = gidx.shape
    nsub = 16
    ncores = 2
    pt = epad // nsub
    nb = pt // 128
    half = n // ncores
    nch = half // r_chunk
    chunk_words = r_chunk * n
    slice_words = chunk_words // nsub
    zwords = 8192
    mesh = plsc.VectorSubcoreMesh(core_axis_name="c", subcore_axis_name="s")

    @functools.partial(
        pl.kernel, mesh=mesh,
        out_type=jax.ShapeDtypeStruct((ne, n * n), jnp.float32),
        scratch_types=[
            pltpu.VMEM((pt,), jnp.int32),
            pltpu.VMEM((pt,), jnp.float32),
            pltpu.VMEM((nb, 128), jnp.int32),
            pltpu.VMEM((nb, 128), jnp.float32),
            pltpu.VMEM((zwords,), jnp.float32),
            pltpu.VMEM_SHARED((chunk_words,), jnp.float32),
            pltpu.SemaphoreType.DMA,
        ],
    )
    def scat(gidx_hbm, val_hbm, out_hbm, gidx_v, val_v,
             idx2d, sval2d, zeros_v, shared, sem):
        cid = lax.axis_index("c")
        sid = lax.axis_index("s")

        def zinit(i, carry):
            zeros_v[pl.ds(i * 16, 16)] = jnp.zeros((16,), jnp.float32)
            return carry
        lax.fori_loop(0, zwords // 16, zinit, 0)

        def plane_body(e, carry):
            pltpu.sync_copy(gidx_hbm.at[e, pl.ds(sid * pt, pt)], gidx_v)
            pltpu.sync_copy(val_hbm.at[e, pl.ds(sid * pt, pt)], val_v)

            def chunk_body(ch, carry2):
                base = (cid * half + ch * r_chunk) * n
                zcopies = [
                    pltpu.async_copy(
                        zeros_v,
                        shared.at[pl.ds(sid * slice_words + z * zwords,
                                        zwords)], sem)
                    for z in range(slice_words // zwords)]
                for zc in zcopies:
                    zc.wait()
                plsc.subcore_barrier()

                def scan_body(b, carry3):
                    for g in range(8):
                        p = b * 128 + g * 16
                        lr = gidx_v[pl.ds(p, 16)] - base
                        v = val_v[pl.ds(p, 16)]
                        inr = (lr >= 0) & (lr < chunk_words)
                        idx2d[b, pl.ds(g * 16, 16)] = lr & (chunk_words - 1)
                        sval2d[b, pl.ds(g * 16, 16)] = jnp.where(inr, v, 0.0)
                    return carry3
                lax.fori_loop(0, nb, scan_body, 0)

                copies = [pltpu.async_copy(sval2d.at[b],
                                           shared.at[idx2d.at[b]],
                                           sem, add=True)
                          for b in range(nb)]
                for cpy in copies:
                    cpy.wait()
                plsc.subcore_barrier()

                pltpu.sync_copy(shared.at[pl.ds(sid * slice_words,
                                                slice_words)],
                                out_hbm.at[e, pl.ds(base + sid * slice_words,
                                                    slice_words)])
                return carry2
            lax.fori_loop(0, nch, chunk_body, 0)
            return carry
        lax.fori_loop(0, ne, plane_body, 0)

    return scat(gidx, val)


def _combine_body(w_ref, a_ref, ra_ref, rb_ref, rb2_ref, *, ne, nc):
    a = a_ref[...]
    for x, oref in enumerate((ra_ref, rb_ref, rb2_ref)):
        for c in range(nc):
            acc = a[0] * w_ref[x, c, 0]
            for e in range(1, ne):
                acc = acc + a[e] * w_ref[x, c, e]
            oref[c] = acc.astype(oref.dtype)


def _combine(w, A, bi=256, bj=2048):
    ne, n, _ = A.shape
    nc = w.shape[1]
    bi = min(bi, n)
    bj = min(bj, n)
    grid = (n // bi, n // bj)
    out_sd = jax.ShapeDtypeStruct((nc, n, n), jnp.bfloat16)
    out_spec = pl.BlockSpec((nc, bi, bj), lambda i, j: (0, i, j))
    return pl.pallas_call(
        functools.partial(_combine_body, ne=ne, nc=nc),
        grid=grid,
        in_specs=[
            pl.BlockSpec(memory_space=pltpu.SMEM),
            pl.BlockSpec((ne, bi, bj), lambda i, j: (0, i, j)),
        ],
        out_specs=[out_spec, out_spec, out_spec],
        out_shape=[out_sd, out_sd, out_sd],
    )(w, A)


def _mm_body(lhs_ref, rhs_ref, out_ref, rs_ref, *, bi, bk, nk, thr):
    c, i, j = pl.program_id(0), pl.program_id(1), pl.program_id(2)
    acc = jnp.dot(lhs_ref[0], rhs_ref[0], preferred_element_type=jnp.float32)
    h = jnp.where(acc > thr, acc, 0.0)
    out_ref[0] = h.astype(out_ref.dtype)

    @pl.when((c == 0) & (i == 0) & (j == 0))
    def _():
        rs_ref[...] = jnp.zeros_like(rs_ref)

    ones = jnp.ones((h.shape[1],), jnp.float32)
    rs_ref[pl.ds(c, 1), pl.ds(i * bi, bi)] += jnp.dot(
        h, ones, preferred_element_type=jnp.float32)[None, :]


def _mm_scaled_body(d_ref, lhs_ref, rhs_ref, out_ref, rs_ref, *, bi, bk, nk,
                    thr):
    c, i, j = pl.program_id(0), pl.program_id(1), pl.program_id(2)

    acc = jnp.dot(lhs_ref[0], rhs_ref[0], preferred_element_type=jnp.float32)
    d = d_ref[pl.ds(c, 1), pl.ds(i * bi, bi)][0]
    dinv = jnp.where(d == 0.0, 0.0, 1.0 / jnp.where(d == 0.0, 1.0, d))
    acc = acc * dinv[:, None]
    h = jnp.where(acc > thr, acc, 0.0)
    out_ref[0] = h.astype(out_ref.dtype)

    @pl.when((c == 0) & (i == 0) & (j == 0))
    def _():
        rs_ref[...] = jnp.zeros_like(rs_ref)

    ones = jnp.ones((h.shape[1],), jnp.float32)
    rs_ref[pl.ds(c, 1), pl.ds(i * bi, bi)] += jnp.dot(
        h, ones, preferred_element_type=jnp.float32)[None, :]


def _mm_threshold(lhs, rhs, d=None, bi=1024, bj=1024, bk=512):
    nc, n, _ = lhs.shape
    bi, bj, bk = min(bi, n), min(bj, n), min(bk, n)
    grid = (nc, n // bi, n // bj)
    nk = n // bk
    out_shape = [jax.ShapeDtypeStruct((nc, n, n), jnp.bfloat16),
                 jax.ShapeDtypeStruct((nc, n), jnp.float32)]
    out_specs = [pl.BlockSpec((1, bi, bj), lambda c, i, j: (c, i, j)),
                 pl.BlockSpec((nc, n), lambda c, i, j: (0, 0))]
    lhs_spec = pl.BlockSpec((1, bi, n), lambda c, i, j: (c, i, 0))
    rhs_spec = pl.BlockSpec((1, n, bj), lambda c, i, j: (c, 0, j))
    if d is None:
        return pl.pallas_call(
            functools.partial(_mm_body, bi=bi, bk=bk, nk=nk, thr=THR),
            grid=grid,
            in_specs=[lhs_spec, rhs_spec],
            out_specs=out_specs,
            out_shape=out_shape,
        )(lhs, rhs)
    d_spec = pl.BlockSpec((nc, n), lambda c, i, j: (0, 0))
    return pl.pallas_call(
        functools.partial(_mm_scaled_body, bi=bi, bk=bk, nk=nk, thr=THR),
        grid=grid,
        in_specs=[d_spec, lhs_spec, rhs_spec],
        out_specs=out_specs,
        out_shape=out_shape,
    )(d, lhs, rhs)


def _gcn_body(d_ref, lhs_ref, xw_ref, b_ref, out_ref, *, bi, bk, nk):
    c, i = pl.program_id(0), pl.program_id(1)

    acc = jnp.dot(lhs_ref[0], xw_ref[...],
                  preferred_element_type=jnp.float32)
    d = d_ref[pl.ds(c, 1), pl.ds(i * bi, bi)][0]
    dinv = jnp.where(d == 0.0, 0.0, 1.0 / jnp.where(d == 0.0, 1.0, d))
    acc = acc * dinv[:, None] + b_ref[...]
    out_ref[0] = jnp.maximum(acc, 0.0)


def _gcn(d, H, XW, b, bi=1024, bk=512):
    nc, n, _ = H.shape
    wout = XW.shape[1]
    bi, bk = min(bi, n), min(bk, n)
    grid = (nc, n // bi)
    return pl.pallas_call(
        functools.partial(_gcn_body, bi=bi, bk=bk, nk=n // bk),
        grid=grid,
        in_specs=[
            pl.BlockSpec((nc, n), lambda c, i: (0, 0)),
            pl.BlockSpec((1, bi, n), lambda c, i: (c, i, 0)),
            pl.BlockSpec((n, wout), lambda c, i: (0, 0)),
            pl.BlockSpec((1, wout), lambda c, i: (0, 0)),
        ],
        out_specs=pl.BlockSpec((1, bi, wout), lambda c, i: (c, i, 0)),
        out_shape=jax.ShapeDtypeStruct((nc, n, wout), jnp.float32),
    )(d, H, XW, b.reshape(1, wout))


def _xw_body(x_ref, w_ref, out_ref):
    out_ref[...] = jnp.dot(x_ref[...], w_ref[...],
                           preferred_element_type=jnp.float32
                           ).astype(out_ref.dtype)


def _xw(X, W):
    return pl.pallas_call(
        _xw_body,
        out_shape=jax.ShapeDtypeStruct((X.shape[0], W.shape[1]),
                                       jnp.bfloat16),
    )(X, W)


def kernel(A_edge_index, A_edge_value, X, l0_w1, l0_w2, l1_w1, gcn_w, gcn_b):
    ne, _, E = A_edge_index.shape
    n = X.shape[0]
    wa = jax.nn.softmax(l0_w1, axis=1)
    wb = jax.nn.softmax(l0_w2, axis=1)
    wb2 = jax.nn.softmax(l1_w1, axis=1)
    w = jnp.stack([wa, wb, wb2])

    nsub = 16
    pt = ((E + nsub * 128 - 1) // (nsub * 128)) * 128
    epad = nsub * pt
    padw = ((0, 0), (0, epad - E))
    gidx = A_edge_index[:, 0, :] * n + A_edge_index[:, 1, :]
    gidx = jnp.pad(gidx, padw, constant_values=n * n)
    val = jnp.pad(A_edge_value, padw, constant_values=0.0)
    A = _sc_scatter(gidx.astype(jnp.int32), val, n).reshape(ne, n, n)

    Ra, Rb, Rb2 = _combine(w, A)
    H1, D1 = _mm_threshold(Ra, Rb)
    H2, D2 = _mm_threshold(H1, Rb2, d=D1)
    XW = _xw(X, gcn_w)
    out = _gcn(D2, H2, XW, gcn_b)
    nc, _, wout = out.shape
    return out.transpose(1, 0, 2).reshape(n, nc * wout)
